# Initial kernel scaffold; baseline (speedup 1.0000x reference)
import jax
import jax.numpy as jnp
from jax import lax
from jax.experimental import pallas as pl
from jax.experimental.pallas import tpu as pltpu
from jax.experimental.pallas import tpu_sc as plsc

NU = 25000
NI = 25000
N = 50000
DF = 128
DL = 64
H = 32
E = 800000
NP = 50176
DEAD = N
EP = 819200
NC = 2
NS = 16
TPN = NP // NS

C2 = 1600
EPT2 = EP // (NC * NS)
K2CH = EPT2 // C2

CS = 2048
EPT = EP // NS
SCH = EPT // CS

_mesh = plsc.VectorSubcoreMesh(core_axis_name="c", subcore_axis_name="s")


def _front_body(f_ref, wut_ref, wvt_ref, wpt_ref, id_ref, o_ref):
    f = f_ref[...]
    a = jnp.dot(f, wut_ref[...], preferred_element_type=jnp.float32)
    b = jnp.dot(f, wvt_ref[...], preferred_element_type=jnp.float32)
    z = jnp.dot(a * b, wpt_ref[...], preferred_element_type=jnp.float32)
    z = jnp.sign(z) * jnp.sqrt(jnp.abs(z) + 1e-9)
    n = jnp.sqrt(jnp.sum(z * z, axis=1, keepdims=True))
    z = z / jnp.maximum(n, 1e-12)
    o_ref[...] = id_ref[...] * z


def _front(features, wut, wvt, wpt, id_emb):
    r = 1000
    return pl.pallas_call(
        _front_body,
        grid=(NI // r,),
        in_specs=[
            pl.BlockSpec((r, DF), lambda b: (b, 0)),
            pl.BlockSpec((DF, 32), lambda b: (0, 0)),
            pl.BlockSpec((DF, 32), lambda b: (0, 0)),
            pl.BlockSpec((32, DL), lambda b: (0, 0)),
            pl.BlockSpec((r, DL), lambda b: (b, 0)),
        ],
        out_specs=pl.BlockSpec((r, DL), lambda b: (b, 0)),
        out_shape=jax.ShapeDtypeStruct((NI, DL), jnp.float32),
    )(features, wut, wvt, wpt, id_emb)


def _scale_body(x0_ref, degp_ref, x_ref, y_ref, dis_ref):
    x0 = x0_ref[...]
    n = jnp.sqrt(jnp.sum(x0 * x0, axis=1, keepdims=True))
    x = x0 / jnp.maximum(n, 1e-12)
    x_ref[...] = x
    deg = degp_ref[0, :] + degp_ref[1, :]
    dis = lax.rsqrt(deg)
    dis_ref[...] = dis[:, None]
    y = x * dis[:, None]
    y_ref[0] = y[:, :H]
    y_ref[1] = y[:, H:]


def _scale(x0, degp):
    r = 1024
    return pl.pallas_call(
        _scale_body,
        grid=(NP // r,),
        in_specs=[
            pl.BlockSpec((r, DL), lambda b: (b, 0)),
            pl.BlockSpec((2, r), lambda b: (0, b)),
        ],
        out_specs=[
            pl.BlockSpec((r, DL), lambda b: (b, 0)),
            pl.BlockSpec((2, r, H), lambda b: (0, b, 0)),
            pl.BlockSpec((r, 1), lambda b: (b, 0)),
        ],
        out_shape=[
            jax.ShapeDtypeStruct((NP, DL), jnp.float32),
            jax.ShapeDtypeStruct((2, NP, H), jnp.float32),
            jax.ShapeDtypeStruct((NP, 1), jnp.float32),
        ],
    )(x0, degp)


def _rescale_body(u_ref, dis_ref, y2_ref):
    d = dis_ref[...]
    d2 = d * d
    y2_ref[0] = u_ref[0] * d2
    y2_ref[1] = u_ref[1] * d2


def _rescale(u, dis):
    r = 1024
    return pl.pallas_call(
        _rescale_body,
        grid=(NP // r,),
        in_specs=[
            pl.BlockSpec((2, r, H), lambda b: (0, b, 0)),
            pl.BlockSpec((r, 1), lambda b: (b, 0)),
        ],
        out_specs=pl.BlockSpec((2, r, H), lambda b: (0, b, 0)),
        out_shape=jax.ShapeDtypeStruct((2, NP, H), jnp.float32),
    )(u, dis)


def _combine_body(x_ref, u_ref, u2_ref, dis_ref, o_ref):
    d = dis_ref[...]
    m = jnp.concatenate(
        [u_ref[0] + u2_ref[0], u_ref[1] + u2_ref[1]], axis=1)
    o_ref[...] = x_ref[...] + d * m


def _combine(x, u, u2, dis):
    r = 1024
    return pl.pallas_call(
        _combine_body,
        grid=(NP // r,),
        in_specs=[
            pl.BlockSpec((r, DL), lambda b: (b, 0)),
            pl.BlockSpec((2, r, H), lambda b: (0, b, 0)),
            pl.BlockSpec((2, r, H), lambda b: (0, b, 0)),
            pl.BlockSpec((r, 1), lambda b: (b, 0)),
        ],
        out_specs=pl.BlockSpec((r, DL), lambda b: (b, 0)),
        out_shape=jax.ShapeDtypeStruct((NP, DL), jnp.float32),
    )(x, u, u2, dis)


def _deg_body(row_hbm, col_hbm, z1_hbm, colp_hbm, degp_hbm,
              rowv, colv, colpv, valv, dega):
    cid = lax.axis_index("c")
    sid = lax.axis_index("s")
    wid = sid * NC + cid
    pltpu.sync_copy(z1_hbm, dega.at[pl.ds(sid * TPN, TPN)])
    plsc.subcore_barrier()

    def ch_body(k, car):
        base = wid * EPT2 + k * C2
        pltpu.sync_copy(row_hbm.at[pl.ds(base, C2)], rowv)
        pltpu.sync_copy(col_hbm.at[pl.ds(base, C2)], colv)

        def vbody(i, cc):
            sl = pl.ds(i * 16, 16)
            r = rowv[sl]
            c = colv[sl]
            m = r != c
            colpv[sl] = jnp.where(m, c, DEAD)
            valv[sl] = jnp.where(m, jnp.float32(1.0), jnp.float32(0.0))
            return cc

        lax.fori_loop(0, C2 // 16, vbody, 0)
        pltpu.sync_copy(colpv, colp_hbm.at[pl.ds(base, C2)])
        pltpu.sync_copy(valv, dega.at[rowv], add=True)
        return car

    lax.fori_loop(0, K2CH, ch_body, 0)
    plsc.subcore_barrier()
    pltpu.sync_copy(dega.at[pl.ds(sid * TPN, TPN)],
                    degp_hbm.at[pl.ds(cid * NP + sid * TPN, TPN)])


def _deg(rowp, colp_in, zeros1):
    return pl.kernel(
        _deg_body,
        out_type=(
            jax.ShapeDtypeStruct((EP,), jnp.int32),
            jax.ShapeDtypeStruct((2 * NP,), jnp.float32),
        ),
        mesh=_mesh,
        scratch_types=[
            pltpu.VMEM((C2,), jnp.int32),
            pltpu.VMEM((C2,), jnp.int32),
            pltpu.VMEM((C2,), jnp.int32),
            pltpu.VMEM((C2,), jnp.float32),
            pltpu.VMEM_SHARED((NP,), jnp.float32),
        ],
    )(rowp, colp_in, zeros1)


def _conv_body(y_hbm, row_hbm, colp_hbm, z2_hbm, out_hbm,
               rowv, rowv2, colpv, rows, acc, sem):
    cid = lax.axis_index("c")
    sid = lax.axis_index("s")
    pltpu.sync_copy(z2_hbm, acc.at[pl.ds(sid * TPN, TPN)])
    plsc.subcore_barrier()
    half = cid * NP

    def ch_body(k, car):
        base = sid * EPT + k * CS
        pltpu.sync_copy(row_hbm.at[pl.ds(base, CS)], rowv)
        pltpu.sync_copy(colp_hbm.at[pl.ds(base, CS)], colpv)

        def vbody(i, cc):
            sl = pl.ds(i * 16, 16)
            rowv2[sl] = rowv[sl] + half
            return cc

        lax.fori_loop(0, CS // 16, vbody, 0)
        pltpu.async_copy(y_hbm.at[rowv2], rows, sem).wait()
        pltpu.sync_copy(rows, acc.at[colpv], add=True)
        return car

    lax.fori_loop(0, SCH, ch_body, 0)
    plsc.subcore_barrier()
    pltpu.sync_copy(acc.at[pl.ds(sid * TPN, TPN)],
                    out_hbm.at[pl.ds(cid * NP + sid * TPN, TPN)])


def _conv_call(y, rowp, colx, zeros2):
    return pl.kernel(
        _conv_body,
        out_type=jax.ShapeDtypeStruct((2 * NP, H), jnp.float32),
        mesh=_mesh,
        scratch_types=[
            pltpu.VMEM((CS,), jnp.int32),
            pltpu.VMEM((CS,), jnp.int32),
            pltpu.VMEM((CS,), jnp.int32),
            pltpu.VMEM((CS, H), jnp.float32),
            pltpu.VMEM_SHARED((NP, H), jnp.float32),
            pltpu.SemaphoreType.DMA,
        ],
    )(y, rowp, colx, zeros2)


def kernel(features, edge_index, Wu, Wv, Wproj, ID_emb, preference):
    row = edge_index[0]
    col = edge_index[1]
    rowp = jnp.pad(row, (0, EP - E))
    colp_in = jnp.pad(col, (0, EP - E))

    temp = _front(features, Wu.T, Wv.T, Wproj.T, ID_emb)

    zeros1 = jnp.zeros((TPN,), jnp.float32)
    zeros2 = jnp.zeros((TPN, H), jnp.float32)
    colx, degp = _deg(rowp, colp_in, zeros1)

    x0 = jnp.concatenate([preference, temp], axis=0)
    x0 = jnp.pad(x0, ((0, NP - N), (0, 0)))
    x, y, dis = _scale(x0, degp.reshape(2, NP))

    u = _conv_call(y.reshape(2 * NP, H), rowp, colx, zeros2)
    y2 = _rescale(u.reshape(2, NP, H), dis)
    u2 = _conv_call(y2.reshape(2 * NP, H), rowp, colx, zeros2)

    xhat = _combine(x, u.reshape(2, NP, H), u2.reshape(2, NP, H), dis)
    return (xhat[:N], preference)


# speedup vs baseline: 14.0681x; 14.0681x over previous
import jax
import jax.numpy as jnp
from jax import lax
from jax.experimental import pallas as pl
from jax.experimental.pallas import tpu as pltpu
from jax.experimental.pallas import tpu_sc as plsc

NU = 25000
NI = 25000
N = 50000
DF = 128
DL = 64
H = 32
E = 800000
NP = 50176
DEAD = N
EP = 819200
NC = 2
NS = 16
TPN = NP // NS

C2 = 1600
EPT2 = EP // (NC * NS)
K2CH = EPT2 // C2

CS = 256
EPT = EP // NS
SCH = EPT // CS

_mesh = plsc.VectorSubcoreMesh(core_axis_name="c", subcore_axis_name="s")


def _front_body(f_ref, wut_ref, wvt_ref, wpt_ref, id_ref, o_ref):
    f = f_ref[...]
    a = jnp.dot(f, wut_ref[...], preferred_element_type=jnp.float32)
    b = jnp.dot(f, wvt_ref[...], preferred_element_type=jnp.float32)
    z = jnp.dot(a * b, wpt_ref[...], preferred_element_type=jnp.float32)
    z = jnp.sign(z) * jnp.sqrt(jnp.abs(z) + 1e-9)
    n = jnp.sqrt(jnp.sum(z * z, axis=1, keepdims=True))
    z = z / jnp.maximum(n, 1e-12)
    o_ref[...] = id_ref[...] * z


def _front(features, wut, wvt, wpt, id_emb):
    r = 1000
    return pl.pallas_call(
        _front_body,
        grid=(NI // r,),
        in_specs=[
            pl.BlockSpec((r, DF), lambda b: (b, 0)),
            pl.BlockSpec((DF, 32), lambda b: (0, 0)),
            pl.BlockSpec((DF, 32), lambda b: (0, 0)),
            pl.BlockSpec((32, DL), lambda b: (0, 0)),
            pl.BlockSpec((r, DL), lambda b: (b, 0)),
        ],
        out_specs=pl.BlockSpec((r, DL), lambda b: (b, 0)),
        out_shape=jax.ShapeDtypeStruct((NI, DL), jnp.float32),
    )(features, wut, wvt, wpt, id_emb)


def _scale_body(x0_ref, degp_ref, x_ref, y_ref, dis_ref, dis2_ref):
    x0 = x0_ref[...]
    n = jnp.sqrt(jnp.sum(x0 * x0, axis=1, keepdims=True))
    x = x0 / jnp.maximum(n, 1e-12)
    x_ref[...] = x
    deg = degp_ref[0, :] + degp_ref[1, :]
    dis = lax.rsqrt(deg)
    dis_ref[...] = dis[:, None]
    dis2_ref[...] = jnp.broadcast_to((dis * dis)[:, None], x0.shape[:1] + (H,))
    y = x * dis[:, None]
    y_ref[0] = y[:, :H]
    y_ref[1] = y[:, H:]


def _scale(x0, degp):
    r = 1024
    return pl.pallas_call(
        _scale_body,
        grid=(NP // r,),
        in_specs=[
            pl.BlockSpec((r, DL), lambda b: (b, 0)),
            pl.BlockSpec((2, r), lambda b: (0, b)),
        ],
        out_specs=[
            pl.BlockSpec((r, DL), lambda b: (b, 0)),
            pl.BlockSpec((2, r, H), lambda b: (0, b, 0)),
            pl.BlockSpec((r, 1), lambda b: (b, 0)),
            pl.BlockSpec((r, H), lambda b: (b, 0)),
        ],
        out_shape=[
            jax.ShapeDtypeStruct((NP, DL), jnp.float32),
            jax.ShapeDtypeStruct((2, NP, H), jnp.float32),
            jax.ShapeDtypeStruct((NP, 1), jnp.float32),
            jax.ShapeDtypeStruct((NP, H), jnp.float32),
        ],
    )(x0, degp)


def _combine_body(x_ref, u_ref, dis_ref, o_ref):
    d = dis_ref[...]
    m = jnp.concatenate([u_ref[0], u_ref[1]], axis=1)
    o_ref[...] = x_ref[...] + d * m


def _combine(x, u12, dis):
    r = 1024
    return pl.pallas_call(
        _combine_body,
        grid=(NP // r,),
        in_specs=[
            pl.BlockSpec((r, DL), lambda b: (b, 0)),
            pl.BlockSpec((2, r, H), lambda b: (0, b, 0)),
            pl.BlockSpec((r, 1), lambda b: (b, 0)),
        ],
        out_specs=pl.BlockSpec((r, DL), lambda b: (b, 0)),
        out_shape=jax.ShapeDtypeStruct((NP, DL), jnp.float32),
    )(x, u12, dis)


def _deg_body(row_hbm, col_hbm, z1_hbm, colp_hbm, degp_hbm,
              rowv, colv, colpv, valv, zv1, dega):
    cid = lax.axis_index("c")
    sid = lax.axis_index("s")
    wid = sid * NC + cid
    pltpu.sync_copy(z1_hbm, zv1)
    pltpu.sync_copy(zv1, dega.at[pl.ds(sid * TPN, TPN)])
    plsc.subcore_barrier()

    def ch_body(k, car):
        base = wid * EPT2 + k * C2
        pltpu.sync_copy(row_hbm.at[pl.ds(base, C2)], rowv)
        pltpu.sync_copy(col_hbm.at[pl.ds(base, C2)], colv)

        def vbody(i, cc):
            sl = pl.ds(i * 16, 16)
            r = rowv[sl]
            c = colv[sl]
            m = r != c
            colpv[sl] = jnp.where(m, c, DEAD)
            valv[sl] = jnp.where(m, jnp.float32(1.0), jnp.float32(0.0))
            return cc

        lax.fori_loop(0, C2 // 16, vbody, 0)
        pltpu.sync_copy(colpv, colp_hbm.at[pl.ds(base, C2)])
        pltpu.sync_copy(valv, dega.at[rowv], add=True)
        return car

    lax.fori_loop(0, K2CH, ch_body, 0)
    plsc.subcore_barrier()
    pltpu.sync_copy(dega.at[pl.ds(sid * TPN, TPN)], zv1)
    pltpu.sync_copy(zv1, degp_hbm.at[pl.ds(cid * NP + sid * TPN, TPN)])


def _deg(rowp, colp_in, zeros1):
    return pl.kernel(
        _deg_body,
        out_type=(
            jax.ShapeDtypeStruct((EP,), jnp.int32),
            jax.ShapeDtypeStruct((2 * NP,), jnp.float32),
        ),
        mesh=_mesh,
        scratch_types=[
            pltpu.VMEM((C2,), jnp.int32),
            pltpu.VMEM((C2,), jnp.int32),
            pltpu.VMEM((C2,), jnp.int32),
            pltpu.VMEM((C2,), jnp.float32),
            pltpu.VMEM((TPN,), jnp.float32),
            pltpu.VMEM_SHARED((NP,), jnp.float32),
        ],
        compiler_params=pltpu.CompilerParams(use_tc_tiling_on_sc=False),
    )(rowp, colp_in, zeros1)


ZR = 196


def _conv_body(y_hbm, row_hbm, colp_hbm, dis2_hbm, z2_hbm, out_hbm, y2_hbm,
               rowva, rowvb, rowvc, colpva, colpvb, colpvc,
               rowsa, rowsb, rowsc, acc, gsema, gsemb, gsemc):
    cid = lax.axis_index("c")
    sid = lax.axis_index("s")
    half = cid * NP

    pltpu.sync_copy(z2_hbm, rowsa.at[pl.ds(0, ZR)])

    def zinit(j, car):
        pltpu.sync_copy(rowsa.at[pl.ds(0, ZR)],
                        acc.at[pl.ds(sid * TPN + j * ZR, ZR)])
        return car

    lax.fori_loop(0, TPN // ZR, zinit, 0)
    plsc.subcore_barrier()

    def run_pass(tab_hbm):
        bufs = ((rowva, colpva, rowsa, gsema),
                (rowvb, colpvb, rowsb, gsemb),
                (rowvc, colpvc, rowsc, gsemc))

        def issue(k, rowv, colpv, rows, gsem):
            base = sid * EPT + k * CS
            pltpu.sync_copy(row_hbm.at[pl.ds(base, CS)], rowv)
            pltpu.sync_copy(colp_hbm.at[pl.ds(base, CS)], colpv)

            def vbody(i, cc):
                sl = pl.ds(i * 16, 16)
                rowv[sl] = rowv[sl] + half
                return cc

            lax.fori_loop(0, CS // 16, vbody, 0)
            return pltpu.async_copy(tab_hbm.at[rowv], rows, gsem)

        issue(0, *bufs[0])
        issue(1, *bufs[1])

        def step(k, car):
            for par in range(3):
                @pl.when(lax.rem(k, 3) == par)
                def _():
                    rowv, colpv, rows, gsem = bufs[par]
                    nxt = bufs[(par + 2) % 3]
                    pltpu.make_async_copy(tab_hbm.at[rowv], rows,
                                          gsem).wait()
                    pltpu.sync_copy(rows, acc.at[colpv], add=True)

                    @pl.when(k < SCH - 2)
                    def _():
                        issue(k + 2, *nxt)
            return car

        lax.fori_loop(0, SCH, step, 0)

    run_pass(y_hbm)
    plsc.subcore_barrier()

    def resc(j, car):
        rbase = sid * TPN + j * ZR
        pltpu.sync_copy(acc.at[pl.ds(rbase, ZR)], rowsa.at[pl.ds(0, ZR)])
        pltpu.sync_copy(dis2_hbm.at[pl.ds(rbase, ZR)], rowsb.at[pl.ds(0, ZR)])

        def vmul(t, cc):
            i = t // 2
            c = (t % 2) * 16
            rowsa[i, pl.ds(c, 16)] = (rowsa[i, pl.ds(c, 16)] *
                                      rowsb[i, pl.ds(c, 16)])
            return cc

        lax.fori_loop(0, ZR * 2, vmul, 0)
        pltpu.sync_copy(rowsa.at[pl.ds(0, ZR)],
                        y2_hbm.at[pl.ds(cid * NP + rbase, ZR)])
        return car

    lax.fori_loop(0, TPN // ZR, resc, 0)
    plsc.subcore_barrier()

    run_pass(y2_hbm)
    plsc.subcore_barrier()

    def wout(j, car):
        pltpu.sync_copy(acc.at[pl.ds(sid * TPN + j * ZR, ZR)],
                        rowsa.at[pl.ds(0, ZR)])
        pltpu.sync_copy(rowsa.at[pl.ds(0, ZR)],
                        out_hbm.at[pl.ds(cid * NP + sid * TPN + j * ZR, ZR)])
        return car

    lax.fori_loop(0, TPN // ZR, wout, 0)


def _conv_call(y, rowp, colx, dis2, zeros2):
    return pl.kernel(
        _conv_body,
        out_type=(
            jax.ShapeDtypeStruct((2 * NP, H), jnp.float32),
            jax.ShapeDtypeStruct((2 * NP, H), jnp.float32),
        ),
        mesh=_mesh,
        scratch_types=[
            pltpu.VMEM((CS,), jnp.int32),
            pltpu.VMEM((CS,), jnp.int32),
            pltpu.VMEM((CS,), jnp.int32),
            pltpu.VMEM((CS,), jnp.int32),
            pltpu.VMEM((CS,), jnp.int32),
            pltpu.VMEM((CS,), jnp.int32),
            pltpu.VMEM((CS, H), jnp.float32),
            pltpu.VMEM((CS, H), jnp.float32),
            pltpu.VMEM((CS, H), jnp.float32),
            pltpu.VMEM_SHARED((NP, H), jnp.float32),
            pltpu.SemaphoreType.DMA,
            pltpu.SemaphoreType.DMA,
            pltpu.SemaphoreType.DMA,
        ],
        compiler_params=pltpu.CompilerParams(use_tc_tiling_on_sc=False),
    )(y, rowp, colx, dis2, zeros2)


def kernel(features, edge_index, Wu, Wv, Wproj, ID_emb, preference):
    row = edge_index[0]
    col = edge_index[1]
    rowp = jnp.pad(row, (0, EP - E))
    colp_in = jnp.pad(col, (0, EP - E))

    temp = _front(features, Wu.T, Wv.T, Wproj.T, ID_emb)

    zeros1 = jnp.zeros((TPN,), jnp.float32)
    zeros2 = jnp.zeros((ZR, H), jnp.float32)
    colx, degp = _deg(rowp, colp_in, zeros1)

    x0 = jnp.concatenate([preference, temp], axis=0)
    x0 = jnp.pad(x0, ((0, NP - N), (0, 0)))
    x, y, dis, dis2 = _scale(x0, degp.reshape(2, NP))

    u12, _ = _conv_call(y.reshape(2 * NP, H), rowp, colx, dis2, zeros2)

    xhat = _combine(x, u12.reshape(2, NP, H), dis)
    return (xhat[:N], preference)


# speedup vs baseline: 14.9789x; 1.0647x over previous
import jax
import jax.numpy as jnp
from jax import lax
from jax.experimental import pallas as pl
from jax.experimental.pallas import tpu as pltpu
from jax.experimental.pallas import tpu_sc as plsc

NU = 25000
NI = 25000
N = 50000
DF = 128
DL = 64
H = 32
E = 800000
NP = 50176
DEAD = N
EP = 819200
NC = 2
NS = 16
TPN = NP // NS

C2 = 1600
EPT2 = EP // (NC * NS)
K2CH = EPT2 // C2

CS = 400
EPT = EP // NS
SCH = EPT // CS
NPAIR = SCH // 2

_mesh = plsc.VectorSubcoreMesh(core_axis_name="c", subcore_axis_name="s")


def _front_body(f_ref, wut_ref, wvt_ref, wpt_ref, id_ref, o_ref):
    f = f_ref[...]
    a = jnp.dot(f, wut_ref[...], preferred_element_type=jnp.float32)
    b = jnp.dot(f, wvt_ref[...], preferred_element_type=jnp.float32)
    z = jnp.dot(a * b, wpt_ref[...], preferred_element_type=jnp.float32)
    z = jnp.sign(z) * jnp.sqrt(jnp.abs(z) + 1e-9)
    n = jnp.sqrt(jnp.sum(z * z, axis=1, keepdims=True))
    z = z / jnp.maximum(n, 1e-12)
    o_ref[...] = id_ref[...] * z


def _front(features, wut, wvt, wpt, id_emb):
    r = 1000
    return pl.pallas_call(
        _front_body,
        grid=(NI // r,),
        in_specs=[
            pl.BlockSpec((r, DF), lambda b: (b, 0)),
            pl.BlockSpec((DF, 32), lambda b: (0, 0)),
            pl.BlockSpec((DF, 32), lambda b: (0, 0)),
            pl.BlockSpec((32, DL), lambda b: (0, 0)),
            pl.BlockSpec((r, DL), lambda b: (b, 0)),
        ],
        out_specs=pl.BlockSpec((r, DL), lambda b: (b, 0)),
        out_shape=jax.ShapeDtypeStruct((NI, DL), jnp.float32),
    )(features, wut, wvt, wpt, id_emb)


def _scale_body(x0_ref, degp_ref, x_ref, y_ref, dis_ref, dis2_ref):
    x0 = x0_ref[...]
    n = jnp.sqrt(jnp.sum(x0 * x0, axis=1, keepdims=True))
    x = x0 / jnp.maximum(n, 1e-12)
    x_ref[...] = x
    deg = degp_ref[0, :] + degp_ref[1, :]
    dis = lax.rsqrt(deg)
    dis_ref[...] = dis[:, None]
    dis2_ref[...] = jnp.broadcast_to((dis * dis)[:, None], x0.shape[:1] + (H,))
    y = x * dis[:, None]
    y_ref[0] = y[:, :H]
    y_ref[1] = y[:, H:]


def _scale(x0, degp):
    r = 1024
    return pl.pallas_call(
        _scale_body,
        grid=(NP // r,),
        in_specs=[
            pl.BlockSpec((r, DL), lambda b: (b, 0)),
            pl.BlockSpec((2, r), lambda b: (0, b)),
        ],
        out_specs=[
            pl.BlockSpec((r, DL), lambda b: (b, 0)),
            pl.BlockSpec((2, r, H), lambda b: (0, b, 0)),
            pl.BlockSpec((r, 1), lambda b: (b, 0)),
            pl.BlockSpec((r, H), lambda b: (b, 0)),
        ],
        out_shape=[
            jax.ShapeDtypeStruct((NP, DL), jnp.float32),
            jax.ShapeDtypeStruct((2, NP, H), jnp.float32),
            jax.ShapeDtypeStruct((NP, 1), jnp.float32),
            jax.ShapeDtypeStruct((NP, H), jnp.float32),
        ],
    )(x0, degp)


def _combine_body(x_ref, u_ref, dis_ref, o_ref):
    d = dis_ref[...]
    m = jnp.concatenate([u_ref[0], u_ref[1]], axis=1)
    o_ref[...] = x_ref[...] + d * m


def _combine(x, u12, dis):
    r = 1024
    return pl.pallas_call(
        _combine_body,
        grid=(NP // r,),
        in_specs=[
            pl.BlockSpec((r, DL), lambda b: (b, 0)),
            pl.BlockSpec((2, r, H), lambda b: (0, b, 0)),
            pl.BlockSpec((r, 1), lambda b: (b, 0)),
        ],
        out_specs=pl.BlockSpec((r, DL), lambda b: (b, 0)),
        out_shape=jax.ShapeDtypeStruct((NP, DL), jnp.float32),
    )(x, u12, dis)


def _deg_body(row_hbm, col_hbm, z1_hbm, colp_hbm, degp_hbm,
              rowv, colv, colpv, valv, zv1, dega):
    cid = lax.axis_index("c")
    sid = lax.axis_index("s")
    wid = sid * NC + cid
    pltpu.sync_copy(z1_hbm, zv1)
    pltpu.sync_copy(zv1, dega.at[pl.ds(sid * TPN, TPN)])
    plsc.subcore_barrier()

    def ch_body(k, car):
        base = wid * EPT2 + k * C2
        pltpu.sync_copy(row_hbm.at[pl.ds(base, C2)], rowv)
        pltpu.sync_copy(col_hbm.at[pl.ds(base, C2)], colv)

        def vbody(i, cc):
            sl = pl.ds(i * 16, 16)
            r = rowv[sl]
            c = colv[sl]
            m = r != c
            colpv[sl] = jnp.where(m, c, DEAD)
            valv[sl] = jnp.where(m, jnp.float32(1.0), jnp.float32(0.0))
            return cc

        lax.fori_loop(0, C2 // 16, vbody, 0)
        pltpu.sync_copy(colpv, colp_hbm.at[pl.ds(base, C2)])
        pltpu.sync_copy(valv, dega.at[rowv], add=True)
        return car

    lax.fori_loop(0, K2CH, ch_body, 0)
    plsc.subcore_barrier()
    pltpu.sync_copy(dega.at[pl.ds(sid * TPN, TPN)], zv1)
    pltpu.sync_copy(zv1, degp_hbm.at[pl.ds(cid * NP + sid * TPN, TPN)])


def _deg(rowp, colp_in, zeros1):
    return pl.kernel(
        _deg_body,
        out_type=(
            jax.ShapeDtypeStruct((EP,), jnp.int32),
            jax.ShapeDtypeStruct((2 * NP,), jnp.float32),
        ),
        mesh=_mesh,
        scratch_types=[
            pltpu.VMEM((C2,), jnp.int32),
            pltpu.VMEM((C2,), jnp.int32),
            pltpu.VMEM((C2,), jnp.int32),
            pltpu.VMEM((C2,), jnp.float32),
            pltpu.VMEM((TPN,), jnp.float32),
            pltpu.VMEM_SHARED((NP,), jnp.float32),
        ],
        compiler_params=pltpu.CompilerParams(use_tc_tiling_on_sc=False),
    )(rowp, colp_in, zeros1)


ZR = 196


def _conv_body(y_hbm, row_hbm, colp_hbm, dis2_hbm, z2_hbm, out_hbm, y2_hbm,
               rowva, rowvb, colpva, colpvb, rowsa, rowsb, acc,
               sema, semb):
    cid = lax.axis_index("c")
    sid = lax.axis_index("s")
    half = cid * NP

    pltpu.sync_copy(z2_hbm, rowsa.at[pl.ds(0, ZR)])

    def zinit(j, car):
        pltpu.sync_copy(rowsa.at[pl.ds(0, ZR)],
                        acc.at[pl.ds(sid * TPN + j * ZR, ZR)])
        return car

    lax.fori_loop(0, TPN // ZR, zinit, 0)
    plsc.subcore_barrier()

    def run_pass(tab_hbm):
        def issue(k, rowv, colpv, rows, sem):
            base = sid * EPT + k * CS
            pltpu.sync_copy(row_hbm.at[pl.ds(base, CS)], rowv)
            pltpu.sync_copy(colp_hbm.at[pl.ds(base, CS)], colpv)

            def vbody(i, cc):
                sl = pl.ds(i * 16, 16)
                rowv[sl] = rowv[sl] + half
                return cc

            lax.fori_loop(0, CS // 16, vbody, 0)
            return pltpu.async_copy(tab_hbm.at[rowv], rows, sem)

        issue(0, rowva, colpva, rowsa, sema)

        def pair(p, car):
            k = p * 2
            db = issue(k + 1, rowvb, colpvb, rowsb, semb)
            pltpu.make_async_copy(tab_hbm.at[rowva], rowsa, sema).wait()
            pltpu.sync_copy(rowsa, acc.at[colpva], add=True)

            @pl.when(p < NPAIR - 1)
            def _():
                issue(k + 2, rowva, colpva, rowsa, sema)

            db.wait()
            pltpu.sync_copy(rowsb, acc.at[colpvb], add=True)
            return car

        lax.fori_loop(0, NPAIR, pair, 0)

    run_pass(y_hbm)
    plsc.subcore_barrier()

    def resc(j, car):
        rbase = sid * TPN + j * ZR
        pltpu.sync_copy(acc.at[pl.ds(rbase, ZR)], rowsa.at[pl.ds(0, ZR)])
        pltpu.sync_copy(dis2_hbm.at[pl.ds(rbase, ZR)], rowsb.at[pl.ds(0, ZR)])

        def vmul(t, cc):
            i = t // 2
            c = (t % 2) * 16
            rowsa[i, pl.ds(c, 16)] = (rowsa[i, pl.ds(c, 16)] *
                                      rowsb[i, pl.ds(c, 16)])
            return cc

        lax.fori_loop(0, ZR * 2, vmul, 0)
        pltpu.sync_copy(rowsa.at[pl.ds(0, ZR)],
                        y2_hbm.at[pl.ds(cid * NP + rbase, ZR)])
        return car

    lax.fori_loop(0, TPN // ZR, resc, 0)
    plsc.subcore_barrier()

    run_pass(y2_hbm)
    plsc.subcore_barrier()

    def wout(j, car):
        pltpu.sync_copy(acc.at[pl.ds(sid * TPN + j * ZR, ZR)],
                        rowsa.at[pl.ds(0, ZR)])
        pltpu.sync_copy(rowsa.at[pl.ds(0, ZR)],
                        out_hbm.at[pl.ds(cid * NP + sid * TPN + j * ZR, ZR)])
        return car

    lax.fori_loop(0, TPN // ZR, wout, 0)


def _conv_call(y, rowp, colx, dis2, zeros2):
    return pl.kernel(
        _conv_body,
        out_type=(
            jax.ShapeDtypeStruct((2 * NP, H), jnp.float32),
            jax.ShapeDtypeStruct((2 * NP, H), jnp.float32),
        ),
        mesh=_mesh,
        scratch_types=[
            pltpu.VMEM((CS,), jnp.int32),
            pltpu.VMEM((CS,), jnp.int32),
            pltpu.VMEM((CS,), jnp.int32),
            pltpu.VMEM((CS,), jnp.int32),
            pltpu.VMEM((CS, H), jnp.float32),
            pltpu.VMEM((CS, H), jnp.float32),
            pltpu.VMEM_SHARED((NP, H), jnp.float32),
            pltpu.SemaphoreType.DMA,
            pltpu.SemaphoreType.DMA,
        ],
        compiler_params=pltpu.CompilerParams(use_tc_tiling_on_sc=False),
    )(y, rowp, colx, dis2, zeros2)


def kernel(features, edge_index, Wu, Wv, Wproj, ID_emb, preference):
    row = edge_index[0]
    col = edge_index[1]
    rowp = jnp.pad(row, (0, EP - E))
    colp_in = jnp.pad(col, (0, EP - E))

    temp = _front(features, Wu.T, Wv.T, Wproj.T, ID_emb)

    zeros1 = jnp.zeros((TPN,), jnp.float32)
    zeros2 = jnp.zeros((ZR, H), jnp.float32)
    colx, degp = _deg(rowp, colp_in, zeros1)

    x0 = jnp.concatenate([preference, temp], axis=0)
    x0 = jnp.pad(x0, ((0, NP - N), (0, 0)))
    x, y, dis, dis2 = _scale(x0, degp.reshape(2, NP))

    u12, _ = _conv_call(y.reshape(2 * NP, H), rowp, colx, dis2, zeros2)

    xhat = _combine(x, u12.reshape(2, NP, H), dis)
    return (xhat[:N], preference)


# speedup vs baseline: 15.0456x; 1.0045x over previous
import jax
import jax.numpy as jnp
from jax import lax
from jax.experimental import pallas as pl
from jax.experimental.pallas import tpu as pltpu
from jax.experimental.pallas import tpu_sc as plsc

NU = 25000
NI = 25000
N = 50000
DF = 128
DL = 64
H = 32
E = 800000
NP = 50176
DEAD = N
EP = 819200
NC = 2
NS = 16
TPN = NP // NS

C2 = 1600
EPT2 = EP // (NC * NS)
K2CH = EPT2 // C2

CS = 400
EPT = EP // NS
SCH = EPT // CS
NPAIR = SCH // 2

_mesh = plsc.VectorSubcoreMesh(core_axis_name="c", subcore_axis_name="s")


def _front_body(f_ref, wut_ref, wvt_ref, wpt_ref, id_ref, o_ref):
    f = f_ref[...]
    a = jnp.dot(f, wut_ref[...], preferred_element_type=jnp.float32)
    b = jnp.dot(f, wvt_ref[...], preferred_element_type=jnp.float32)
    z = jnp.dot(a * b, wpt_ref[...], preferred_element_type=jnp.float32)
    z = jnp.sign(z) * jnp.sqrt(jnp.abs(z) + 1e-9)
    n = jnp.sqrt(jnp.sum(z * z, axis=1, keepdims=True))
    z = z / jnp.maximum(n, 1e-12)
    o_ref[...] = id_ref[...] * z


def _front(features, wut, wvt, wpt, id_emb):
    r = 1000
    return pl.pallas_call(
        _front_body,
        grid=(NI // r,),
        in_specs=[
            pl.BlockSpec((r, DF), lambda b: (b, 0)),
            pl.BlockSpec((DF, 32), lambda b: (0, 0)),
            pl.BlockSpec((DF, 32), lambda b: (0, 0)),
            pl.BlockSpec((32, DL), lambda b: (0, 0)),
            pl.BlockSpec((r, DL), lambda b: (b, 0)),
        ],
        out_specs=pl.BlockSpec((r, DL), lambda b: (b, 0)),
        out_shape=jax.ShapeDtypeStruct((NI, DL), jnp.float32),
    )(features, wut, wvt, wpt, id_emb)


def _scale_body(x0_ref, degp_ref, x_ref, y_ref, dis_ref, dis2_ref):
    x0 = x0_ref[...]
    n = jnp.sqrt(jnp.sum(x0 * x0, axis=1, keepdims=True))
    x = x0 / jnp.maximum(n, 1e-12)
    x_ref[...] = x
    deg = degp_ref[0, :] + degp_ref[1, :]
    dis = lax.rsqrt(deg)
    dis_ref[...] = dis[:, None]
    dis2_ref[...] = jnp.broadcast_to((dis * dis)[:, None], x0.shape[:1] + (H,))
    y = x * dis[:, None]
    y_ref[0] = y[:, :H]
    y_ref[1] = y[:, H:]


def _scale(x0, degp):
    r = 1024
    return pl.pallas_call(
        _scale_body,
        grid=(NP // r,),
        in_specs=[
            pl.BlockSpec((r, DL), lambda b: (b, 0)),
            pl.BlockSpec((2, r), lambda b: (0, b)),
        ],
        out_specs=[
            pl.BlockSpec((r, DL), lambda b: (b, 0)),
            pl.BlockSpec((2, r, H), lambda b: (0, b, 0)),
            pl.BlockSpec((r, 1), lambda b: (b, 0)),
            pl.BlockSpec((r, H), lambda b: (b, 0)),
        ],
        out_shape=[
            jax.ShapeDtypeStruct((NP, DL), jnp.float32),
            jax.ShapeDtypeStruct((2, NP, H), jnp.float32),
            jax.ShapeDtypeStruct((NP, 1), jnp.float32),
            jax.ShapeDtypeStruct((NP, H), jnp.float32),
        ],
    )(x0, degp)


def _combine_body(x_ref, u_ref, dis_ref, o_ref):
    d = dis_ref[...]
    m = jnp.concatenate([u_ref[0], u_ref[1]], axis=1)
    o_ref[...] = x_ref[...] + d * m


def _combine(x, u12, dis):
    r = 1024
    return pl.pallas_call(
        _combine_body,
        grid=(NP // r,),
        in_specs=[
            pl.BlockSpec((r, DL), lambda b: (b, 0)),
            pl.BlockSpec((2, r, H), lambda b: (0, b, 0)),
            pl.BlockSpec((r, 1), lambda b: (b, 0)),
        ],
        out_specs=pl.BlockSpec((r, DL), lambda b: (b, 0)),
        out_shape=jax.ShapeDtypeStruct((NP, DL), jnp.float32),
    )(x, u12, dis)


def _deg_body(row_hbm, col_hbm, z1_hbm, colp_hbm, degp_hbm,
              rowv, colv, colpv, valv, zv1, dega):
    cid = lax.axis_index("c")
    sid = lax.axis_index("s")
    wid = sid * NC + cid
    pltpu.sync_copy(z1_hbm, zv1)
    pltpu.sync_copy(zv1, dega.at[pl.ds(sid * TPN, TPN)])
    plsc.subcore_barrier()

    def ch_body(k, car):
        base = wid * EPT2 + k * C2
        pltpu.sync_copy(row_hbm.at[pl.ds(base, C2)], rowv)
        pltpu.sync_copy(col_hbm.at[pl.ds(base, C2)], colv)

        def vbody(i, cc):
            sl = pl.ds(i * 16, 16)
            r = rowv[sl]
            c = colv[sl]
            m = r != c
            colpv[sl] = jnp.where(m, c, DEAD)
            valv[sl] = jnp.where(m, jnp.float32(1.0), jnp.float32(0.0))
            return cc

        lax.fori_loop(0, C2 // 16, vbody, 0)
        pltpu.sync_copy(colpv, colp_hbm.at[pl.ds(base, C2)])
        pltpu.sync_copy(valv, dega.at[rowv], add=True)
        return car

    lax.fori_loop(0, K2CH, ch_body, 0)
    plsc.subcore_barrier()
    pltpu.sync_copy(dega.at[pl.ds(sid * TPN, TPN)], zv1)
    pltpu.sync_copy(zv1, degp_hbm.at[pl.ds(cid * NP + sid * TPN, TPN)])


def _deg(rowp, colp_in, zeros1):
    return pl.kernel(
        _deg_body,
        out_type=(
            jax.ShapeDtypeStruct((EP,), jnp.int32),
            jax.ShapeDtypeStruct((2 * NP,), jnp.float32),
        ),
        mesh=_mesh,
        scratch_types=[
            pltpu.VMEM((C2,), jnp.int32),
            pltpu.VMEM((C2,), jnp.int32),
            pltpu.VMEM((C2,), jnp.int32),
            pltpu.VMEM((C2,), jnp.float32),
            pltpu.VMEM((TPN,), jnp.float32),
            pltpu.VMEM_SHARED((NP,), jnp.float32),
        ],
        compiler_params=pltpu.CompilerParams(use_tc_tiling_on_sc=False),
    )(rowp, colp_in, zeros1)


ZR = 392


def _conv_body(y_hbm, row_hbm, colp_hbm, dis2_hbm, z2_hbm, out_hbm, y2_hbm,
               rowva, rowvb, colpva, colpvb, rowsa, rowsb, acc,
               sema, semb):
    cid = lax.axis_index("c")
    sid = lax.axis_index("s")
    half = cid * NP

    pltpu.sync_copy(z2_hbm, rowsa.at[pl.ds(0, ZR)])

    def zinit(j, car):
        pltpu.sync_copy(rowsa.at[pl.ds(0, ZR)],
                        acc.at[pl.ds(sid * TPN + j * ZR, ZR)])
        return car

    lax.fori_loop(0, TPN // ZR, zinit, 0)
    plsc.subcore_barrier()

    def run_pass(tab_hbm):
        def issue(k, rowv, colpv, rows, sem):
            base = sid * EPT + k * CS
            pltpu.sync_copy(row_hbm.at[pl.ds(base, CS)], rowv)
            pltpu.sync_copy(colp_hbm.at[pl.ds(base, CS)], colpv)

            def vbody(i, cc):
                sl = pl.ds(i * 16, 16)
                rowv[sl] = rowv[sl] + half
                return cc

            lax.fori_loop(0, CS // 16, vbody, 0)
            return pltpu.async_copy(tab_hbm.at[rowv], rows, sem)

        issue(0, rowva, colpva, rowsa, sema)

        def pair(p, car):
            k = p * 2
            db = issue(k + 1, rowvb, colpvb, rowsb, semb)
            pltpu.make_async_copy(tab_hbm.at[rowva], rowsa, sema).wait()
            pltpu.sync_copy(rowsa, acc.at[colpva], add=True)

            @pl.when(p < NPAIR - 1)
            def _():
                issue(k + 2, rowva, colpva, rowsa, sema)

            db.wait()
            pltpu.sync_copy(rowsb, acc.at[colpvb], add=True)
            return car

        lax.fori_loop(0, NPAIR, pair, 0)

    run_pass(y_hbm)
    plsc.subcore_barrier()

    def resc(j, car):
        rbase = sid * TPN + j * ZR
        pltpu.sync_copy(acc.at[pl.ds(rbase, ZR)], rowsa.at[pl.ds(0, ZR)])
        pltpu.sync_copy(dis2_hbm.at[pl.ds(rbase, ZR)], rowsb.at[pl.ds(0, ZR)])

        def vmul(t, cc):
            i = t // 2
            c = (t % 2) * 16
            rowsa[i, pl.ds(c, 16)] = (rowsa[i, pl.ds(c, 16)] *
                                      rowsb[i, pl.ds(c, 16)])
            return cc

        lax.fori_loop(0, ZR * 2, vmul, 0)
        pltpu.sync_copy(rowsa.at[pl.ds(0, ZR)],
                        y2_hbm.at[pl.ds(cid * NP + rbase, ZR)])
        return car

    lax.fori_loop(0, TPN // ZR, resc, 0)
    plsc.subcore_barrier()

    run_pass(y2_hbm)
    plsc.subcore_barrier()

    def wout(j, car):
        pltpu.sync_copy(acc.at[pl.ds(sid * TPN + j * ZR, ZR)],
                        rowsa.at[pl.ds(0, ZR)])
        pltpu.sync_copy(rowsa.at[pl.ds(0, ZR)],
                        out_hbm.at[pl.ds(cid * NP + sid * TPN + j * ZR, ZR)])
        return car

    lax.fori_loop(0, TPN // ZR, wout, 0)


def _conv_call(y, rowp, colx, dis2, zeros2):
    return pl.kernel(
        _conv_body,
        out_type=(
            jax.ShapeDtypeStruct((2 * NP, H), jnp.float32),
            jax.ShapeDtypeStruct((2 * NP, H), jnp.float32),
        ),
        mesh=_mesh,
        scratch_types=[
            pltpu.VMEM((CS,), jnp.int32),
            pltpu.VMEM((CS,), jnp.int32),
            pltpu.VMEM((CS,), jnp.int32),
            pltpu.VMEM((CS,), jnp.int32),
            pltpu.VMEM((CS, H), jnp.float32),
            pltpu.VMEM((CS, H), jnp.float32),
            pltpu.VMEM_SHARED((NP, H), jnp.float32),
            pltpu.SemaphoreType.DMA,
            pltpu.SemaphoreType.DMA,
        ],
        compiler_params=pltpu.CompilerParams(use_tc_tiling_on_sc=False),
    )(y, rowp, colx, dis2, zeros2)


def kernel(features, edge_index, Wu, Wv, Wproj, ID_emb, preference):
    row = edge_index[0]
    col = edge_index[1]
    rowp = jnp.pad(row, (0, EP - E))
    colp_in = jnp.pad(col, (0, EP - E))

    temp = _front(features, Wu.T, Wv.T, Wproj.T, ID_emb)

    zeros1 = jnp.zeros((TPN,), jnp.float32)
    zeros2 = jnp.zeros((ZR, H), jnp.float32)
    colx, degp = _deg(rowp, colp_in, zeros1)

    x0 = jnp.concatenate([preference, temp], axis=0)
    x0 = jnp.pad(x0, ((0, NP - N), (0, 0)))
    x, y, dis, dis2 = _scale(x0, degp.reshape(2, NP))

    u12, _ = _conv_call(y.reshape(2 * NP, H), rowp, colx, dis2, zeros2)

    xhat = _combine(x, u12.reshape(2, NP, H), dis)
    return (xhat[:N], preference)


# speedup vs baseline: 15.4711x; 1.0283x over previous
import jax
import jax.numpy as jnp
from jax import lax
from jax.experimental import pallas as pl
from jax.experimental.pallas import tpu as pltpu
from jax.experimental.pallas import tpu_sc as plsc

NU = 25000
NI = 25000
N = 50000
DF = 128
DL = 64
H = 32
E = 800000
NP = 50176
DEAD = N
EP = 819200
NC = 2
NS = 16
TPN = NP // NS

C2 = 1600
EPT2 = EP // (NC * NS)
K2CH = EPT2 // C2

CS = 400
EPT = EP // NS
SCH = EPT // CS
NPAIR = SCH // 2

_mesh = plsc.VectorSubcoreMesh(core_axis_name="c", subcore_axis_name="s")


def _front_body(f_ref, wut_ref, wvt_ref, wpt_ref, id_ref, o_ref):
    f = f_ref[...]
    a = jnp.dot(f, wut_ref[...], preferred_element_type=jnp.float32)
    b = jnp.dot(f, wvt_ref[...], preferred_element_type=jnp.float32)
    z = jnp.dot(a * b, wpt_ref[...], preferred_element_type=jnp.float32)
    z = jnp.sign(z) * jnp.sqrt(jnp.abs(z) + 1e-9)
    n = jnp.sqrt(jnp.sum(z * z, axis=1, keepdims=True))
    z = z / jnp.maximum(n, 1e-12)
    o_ref[...] = id_ref[...] * z


def _front(features, wut, wvt, wpt, id_emb):
    r = 5000
    return pl.pallas_call(
        _front_body,
        grid=(NI // r,),
        in_specs=[
            pl.BlockSpec((r, DF), lambda b: (b, 0)),
            pl.BlockSpec((DF, 32), lambda b: (0, 0)),
            pl.BlockSpec((DF, 32), lambda b: (0, 0)),
            pl.BlockSpec((32, DL), lambda b: (0, 0)),
            pl.BlockSpec((r, DL), lambda b: (b, 0)),
        ],
        out_specs=pl.BlockSpec((r, DL), lambda b: (b, 0)),
        out_shape=jax.ShapeDtypeStruct((NI, DL), jnp.float32),
    )(features, wut, wvt, wpt, id_emb)


def _scale_body(x0_ref, degp_ref, x_ref, y_ref, dis_ref, dis2_ref):
    x0 = x0_ref[...]
    n = jnp.sqrt(jnp.sum(x0 * x0, axis=1, keepdims=True))
    x = x0 / jnp.maximum(n, 1e-12)
    x_ref[...] = x
    deg = degp_ref[0, :] + degp_ref[1, :]
    dis = lax.rsqrt(deg)
    dis_ref[...] = dis[:, None]
    dis2_ref[...] = jnp.broadcast_to((dis * dis)[:, None], x0.shape[:1] + (H,))
    y = x * dis[:, None]
    y_ref[0] = y[:, :H]
    y_ref[1] = y[:, H:]


def _scale(x0, degp):
    r = 3584
    return pl.pallas_call(
        _scale_body,
        grid=(NP // r,),
        in_specs=[
            pl.BlockSpec((r, DL), lambda b: (b, 0)),
            pl.BlockSpec((2, r), lambda b: (0, b)),
        ],
        out_specs=[
            pl.BlockSpec((r, DL), lambda b: (b, 0)),
            pl.BlockSpec((2, r, H), lambda b: (0, b, 0)),
            pl.BlockSpec((r, 1), lambda b: (b, 0)),
            pl.BlockSpec((r, H), lambda b: (b, 0)),
        ],
        out_shape=[
            jax.ShapeDtypeStruct((NP, DL), jnp.float32),
            jax.ShapeDtypeStruct((2, NP, H), jnp.float32),
            jax.ShapeDtypeStruct((NP, 1), jnp.float32),
            jax.ShapeDtypeStruct((NP, H), jnp.float32),
        ],
    )(x0, degp)


def _combine_body(x_ref, u_ref, dis_ref, o_ref):
    d = dis_ref[...]
    m = jnp.concatenate([u_ref[0], u_ref[1]], axis=1)
    o_ref[...] = x_ref[...] + d * m


def _combine(x, u12, dis):
    r = 3584
    return pl.pallas_call(
        _combine_body,
        grid=(NP // r,),
        in_specs=[
            pl.BlockSpec((r, DL), lambda b: (b, 0)),
            pl.BlockSpec((2, r, H), lambda b: (0, b, 0)),
            pl.BlockSpec((r, 1), lambda b: (b, 0)),
        ],
        out_specs=pl.BlockSpec((r, DL), lambda b: (b, 0)),
        out_shape=jax.ShapeDtypeStruct((NP, DL), jnp.float32),
    )(x, u12, dis)


def _deg_body(row_hbm, col_hbm, z1_hbm, colp_hbm, degp_hbm,
              rowv, colv, colpv, valv, zv1, dega):
    cid = lax.axis_index("c")
    sid = lax.axis_index("s")
    wid = sid * NC + cid
    pltpu.sync_copy(z1_hbm, zv1)
    pltpu.sync_copy(zv1, dega.at[pl.ds(sid * TPN, TPN)])
    plsc.subcore_barrier()

    def ch_body(k, car):
        base = wid * EPT2 + k * C2
        pltpu.sync_copy(row_hbm.at[pl.ds(base, C2)], rowv)
        pltpu.sync_copy(col_hbm.at[pl.ds(base, C2)], colv)

        def vbody(i, cc):
            sl = pl.ds(i * 16, 16)
            r = rowv[sl]
            c = colv[sl]
            m = r != c
            colpv[sl] = jnp.where(m, c, DEAD)
            valv[sl] = jnp.where(m, jnp.float32(1.0), jnp.float32(0.0))
            return cc

        lax.fori_loop(0, C2 // 16, vbody, 0)
        pltpu.sync_copy(colpv, colp_hbm.at[pl.ds(base, C2)])
        pltpu.sync_copy(valv, dega.at[rowv], add=True)
        return car

    lax.fori_loop(0, K2CH, ch_body, 0)
    plsc.subcore_barrier()
    pltpu.sync_copy(dega.at[pl.ds(sid * TPN, TPN)], zv1)
    pltpu.sync_copy(zv1, degp_hbm.at[pl.ds(cid * NP + sid * TPN, TPN)])


def _deg(rowp, colp_in, zeros1):
    return pl.kernel(
        _deg_body,
        out_type=(
            jax.ShapeDtypeStruct((EP,), jnp.int32),
            jax.ShapeDtypeStruct((2 * NP,), jnp.float32),
        ),
        mesh=_mesh,
        scratch_types=[
            pltpu.VMEM((C2,), jnp.int32),
            pltpu.VMEM((C2,), jnp.int32),
            pltpu.VMEM((C2,), jnp.int32),
            pltpu.VMEM((C2,), jnp.float32),
            pltpu.VMEM((TPN,), jnp.float32),
            pltpu.VMEM_SHARED((NP,), jnp.float32),
        ],
        compiler_params=pltpu.CompilerParams(use_tc_tiling_on_sc=False),
    )(rowp, colp_in, zeros1)


ZR = 392


def _conv_body(y_hbm, row_hbm, colp_hbm, dis2_hbm, z2_hbm, out_hbm, y2_hbm,
               rowva, rowvb, colpva, colpvb, rowsa, rowsb, acc,
               sema, semb):
    cid = lax.axis_index("c")
    sid = lax.axis_index("s")
    half = cid * NP

    pltpu.sync_copy(z2_hbm, rowsa.at[pl.ds(0, ZR)])

    def zinit(j, car):
        pltpu.sync_copy(rowsa.at[pl.ds(0, ZR)],
                        acc.at[pl.ds(sid * TPN + j * ZR, ZR)])
        return car

    lax.fori_loop(0, TPN // ZR, zinit, 0)
    plsc.subcore_barrier()

    def run_pass(tab_hbm):
        def issue(k, rowv, colpv, rows, sem):
            base = sid * EPT + k * CS
            pltpu.sync_copy(row_hbm.at[pl.ds(base, CS)], rowv)
            pltpu.sync_copy(colp_hbm.at[pl.ds(base, CS)], colpv)

            def vbody(i, cc):
                sl = pl.ds(i * 16, 16)
                rowv[sl] = rowv[sl] + half
                return cc

            lax.fori_loop(0, CS // 16, vbody, 0)
            return pltpu.async_copy(tab_hbm.at[rowv], rows, sem)

        issue(0, rowva, colpva, rowsa, sema)

        def pair(p, car):
            k = p * 2
            db = issue(k + 1, rowvb, colpvb, rowsb, semb)
            pltpu.make_async_copy(tab_hbm.at[rowva], rowsa, sema).wait()
            pltpu.sync_copy(rowsa, acc.at[colpva], add=True)

            @pl.when(p < NPAIR - 1)
            def _():
                issue(k + 2, rowva, colpva, rowsa, sema)

            db.wait()
            pltpu.sync_copy(rowsb, acc.at[colpvb], add=True)
            return car

        lax.fori_loop(0, NPAIR, pair, 0)

    run_pass(y_hbm)
    plsc.subcore_barrier()

    def resc(j, car):
        rbase = sid * TPN + j * ZR
        pltpu.sync_copy(acc.at[pl.ds(rbase, ZR)], rowsa.at[pl.ds(0, ZR)])
        pltpu.sync_copy(dis2_hbm.at[pl.ds(rbase, ZR)], rowsb.at[pl.ds(0, ZR)])

        def vmul(t, cc):
            i = t // 2
            c = (t % 2) * 16
            rowsa[i, pl.ds(c, 16)] = (rowsa[i, pl.ds(c, 16)] *
                                      rowsb[i, pl.ds(c, 16)])
            return cc

        lax.fori_loop(0, ZR * 2, vmul, 0)
        pltpu.sync_copy(rowsa.at[pl.ds(0, ZR)],
                        y2_hbm.at[pl.ds(cid * NP + rbase, ZR)])
        return car

    lax.fori_loop(0, TPN // ZR, resc, 0)
    plsc.subcore_barrier()

    run_pass(y2_hbm)
    plsc.subcore_barrier()

    def wout(j, car):
        pltpu.sync_copy(acc.at[pl.ds(sid * TPN + j * ZR, ZR)],
                        rowsa.at[pl.ds(0, ZR)])
        pltpu.sync_copy(rowsa.at[pl.ds(0, ZR)],
                        out_hbm.at[pl.ds(cid * NP + sid * TPN + j * ZR, ZR)])
        return car

    lax.fori_loop(0, TPN // ZR, wout, 0)


def _conv_call(y, rowp, colx, dis2, zeros2):
    return pl.kernel(
        _conv_body,
        out_type=(
            jax.ShapeDtypeStruct((2 * NP, H), jnp.float32),
            jax.ShapeDtypeStruct((2 * NP, H), jnp.float32),
        ),
        mesh=_mesh,
        scratch_types=[
            pltpu.VMEM((CS,), jnp.int32),
            pltpu.VMEM((CS,), jnp.int32),
            pltpu.VMEM((CS,), jnp.int32),
            pltpu.VMEM((CS,), jnp.int32),
            pltpu.VMEM((CS, H), jnp.float32),
            pltpu.VMEM((CS, H), jnp.float32),
            pltpu.VMEM_SHARED((NP, H), jnp.float32),
            pltpu.SemaphoreType.DMA,
            pltpu.SemaphoreType.DMA,
        ],
        compiler_params=pltpu.CompilerParams(use_tc_tiling_on_sc=False),
    )(y, rowp, colx, dis2, zeros2)


def kernel(features, edge_index, Wu, Wv, Wproj, ID_emb, preference):
    row = edge_index[0]
    col = edge_index[1]
    rowp = jnp.pad(row, (0, EP - E))
    colp_in = jnp.pad(col, (0, EP - E))

    temp = _front(features, Wu.T, Wv.T, Wproj.T, ID_emb)

    zeros1 = jnp.zeros((TPN,), jnp.float32)
    zeros2 = jnp.zeros((ZR, H), jnp.float32)
    colx, degp = _deg(rowp, colp_in, zeros1)

    x0 = jnp.concatenate([preference, temp], axis=0)
    x0 = jnp.pad(x0, ((0, NP - N), (0, 0)))
    x, y, dis, dis2 = _scale(x0, degp.reshape(2, NP))

    u12, _ = _conv_call(y.reshape(2 * NP, H), rowp, colx, dis2, zeros2)

    xhat = _combine(x, u12.reshape(2, NP, H), dis)
    return (xhat[:N], preference)


# speedup vs baseline: 15.4868x; 1.0010x over previous
import jax
import jax.numpy as jnp
from jax import lax
from jax.experimental import pallas as pl
from jax.experimental.pallas import tpu as pltpu
from jax.experimental.pallas import tpu_sc as plsc

NU = 25000
NI = 25000
N = 50000
DF = 128
DL = 64
H = 32
E = 800000
NP = 50176
DEAD = N
EP = 819200
NC = 2
NS = 16
TPN = NP // NS

C2 = 1600
EPT2 = EP // (NC * NS)
K2CH = EPT2 // C2

CS = 400
EPT = EP // NS
SCH = EPT // CS
NPAIR = SCH // 2

_mesh = plsc.VectorSubcoreMesh(core_axis_name="c", subcore_axis_name="s")


def _front_body(f_ref, wut_ref, wvt_ref, wpt_ref, id_ref, o_ref):
    f = f_ref[...]
    a = jnp.dot(f, wut_ref[...], preferred_element_type=jnp.float32)
    b = jnp.dot(f, wvt_ref[...], preferred_element_type=jnp.float32)
    z = jnp.dot(a * b, wpt_ref[...], preferred_element_type=jnp.float32)
    z = jnp.sign(z) * jnp.sqrt(jnp.abs(z) + 1e-9)
    n = jnp.sqrt(jnp.sum(z * z, axis=1, keepdims=True))
    z = z / jnp.maximum(n, 1e-12)
    o_ref[...] = id_ref[...] * z


def _front(features, wut, wvt, wpt, id_emb):
    r = 5000
    return pl.pallas_call(
        _front_body,
        grid=(NI // r,),
        in_specs=[
            pl.BlockSpec((r, DF), lambda b: (b, 0)),
            pl.BlockSpec((DF, 32), lambda b: (0, 0)),
            pl.BlockSpec((DF, 32), lambda b: (0, 0)),
            pl.BlockSpec((32, DL), lambda b: (0, 0)),
            pl.BlockSpec((r, DL), lambda b: (b, 0)),
        ],
        out_specs=pl.BlockSpec((r, DL), lambda b: (b, 0)),
        out_shape=jax.ShapeDtypeStruct((NI, DL), jnp.float32),
    )(features, wut, wvt, wpt, id_emb)


def _scale_body(x0_ref, degp_ref, x_ref, y_ref, dis_ref, dis2_ref):
    x0 = x0_ref[...]
    n = jnp.sqrt(jnp.sum(x0 * x0, axis=1, keepdims=True))
    x = x0 / jnp.maximum(n, 1e-12)
    x_ref[...] = x
    deg = degp_ref[0, :] + degp_ref[1, :]
    dis = lax.rsqrt(deg)
    dis_ref[...] = dis[:, None]
    dis2_ref[...] = jnp.broadcast_to((dis * dis)[:, None], x0.shape[:1] + (H,))
    y = x * dis[:, None]
    y_ref[0] = y[:, :H]
    y_ref[1] = y[:, H:]


def _scale(x0, degp):
    r = 7168
    return pl.pallas_call(
        _scale_body,
        grid=(NP // r,),
        in_specs=[
            pl.BlockSpec((r, DL), lambda b: (b, 0)),
            pl.BlockSpec((2, r), lambda b: (0, b)),
        ],
        out_specs=[
            pl.BlockSpec((r, DL), lambda b: (b, 0)),
            pl.BlockSpec((2, r, H), lambda b: (0, b, 0)),
            pl.BlockSpec((r, 1), lambda b: (b, 0)),
            pl.BlockSpec((r, H), lambda b: (b, 0)),
        ],
        out_shape=[
            jax.ShapeDtypeStruct((NP, DL), jnp.float32),
            jax.ShapeDtypeStruct((2, NP, H), jnp.float32),
            jax.ShapeDtypeStruct((NP, 1), jnp.float32),
            jax.ShapeDtypeStruct((NP, H), jnp.float32),
        ],
    )(x0, degp)


def _combine_body(x_ref, u_ref, dis_ref, o_ref):
    d = dis_ref[...]
    m = jnp.concatenate([u_ref[0], u_ref[1]], axis=1)
    o_ref[...] = x_ref[...] + d * m


def _combine(x, u12, dis):
    r = 7168
    return pl.pallas_call(
        _combine_body,
        grid=(NP // r,),
        in_specs=[
            pl.BlockSpec((r, DL), lambda b: (b, 0)),
            pl.BlockSpec((2, r, H), lambda b: (0, b, 0)),
            pl.BlockSpec((r, 1), lambda b: (b, 0)),
        ],
        out_specs=pl.BlockSpec((r, DL), lambda b: (b, 0)),
        out_shape=jax.ShapeDtypeStruct((NP, DL), jnp.float32),
    )(x, u12, dis)


def _deg_body(row_hbm, col_hbm, z1_hbm, colp_hbm, degp_hbm,
              rowv, colv, colpv, valv, zv1, dega):
    cid = lax.axis_index("c")
    sid = lax.axis_index("s")
    wid = sid * NC + cid
    pltpu.sync_copy(z1_hbm, zv1)
    pltpu.sync_copy(zv1, dega.at[pl.ds(sid * TPN, TPN)])
    plsc.subcore_barrier()

    def ch_body(k, car):
        base = wid * EPT2 + k * C2
        pltpu.sync_copy(row_hbm.at[pl.ds(base, C2)], rowv)
        pltpu.sync_copy(col_hbm.at[pl.ds(base, C2)], colv)

        def vbody(i, cc):
            sl = pl.ds(i * 16, 16)
            r = rowv[sl]
            c = colv[sl]
            m = r != c
            colpv[sl] = jnp.where(m, c, DEAD)
            valv[sl] = jnp.where(m, jnp.float32(1.0), jnp.float32(0.0))
            return cc

        lax.fori_loop(0, C2 // 16, vbody, 0)
        pltpu.sync_copy(colpv, colp_hbm.at[pl.ds(base, C2)])
        pltpu.sync_copy(valv, dega.at[rowv], add=True)
        return car

    lax.fori_loop(0, K2CH, ch_body, 0)
    plsc.subcore_barrier()
    pltpu.sync_copy(dega.at[pl.ds(sid * TPN, TPN)], zv1)
    pltpu.sync_copy(zv1, degp_hbm.at[pl.ds(cid * NP + sid * TPN, TPN)])


def _deg(rowp, colp_in, zeros1):
    return pl.kernel(
        _deg_body,
        out_type=(
            jax.ShapeDtypeStruct((EP,), jnp.int32),
            jax.ShapeDtypeStruct((2 * NP,), jnp.float32),
        ),
        mesh=_mesh,
        scratch_types=[
            pltpu.VMEM((C2,), jnp.int32),
            pltpu.VMEM((C2,), jnp.int32),
            pltpu.VMEM((C2,), jnp.int32),
            pltpu.VMEM((C2,), jnp.float32),
            pltpu.VMEM((TPN,), jnp.float32),
            pltpu.VMEM_SHARED((NP,), jnp.float32),
        ],
        compiler_params=pltpu.CompilerParams(use_tc_tiling_on_sc=False),
    )(rowp, colp_in, zeros1)


ZR = 392


def _conv_body(y_hbm, row_hbm, colp_hbm, dis2_hbm, z2_hbm, out_hbm, y2_hbm,
               rowva, rowvb, colpva, colpvb, rowsa, rowsb, acc,
               sema, semb):
    cid = lax.axis_index("c")
    sid = lax.axis_index("s")
    half = cid * NP

    pltpu.sync_copy(z2_hbm, rowsa.at[pl.ds(0, ZR)])

    def zinit(j, car):
        pltpu.sync_copy(rowsa.at[pl.ds(0, ZR)],
                        acc.at[pl.ds(sid * TPN + j * ZR, ZR)])
        return car

    lax.fori_loop(0, TPN // ZR, zinit, 0)
    plsc.subcore_barrier()

    def run_pass(tab_hbm):
        def issue(k, rowv, colpv, rows, sem):
            base = sid * EPT + k * CS
            pltpu.sync_copy(row_hbm.at[pl.ds(base, CS)], rowv)
            pltpu.sync_copy(colp_hbm.at[pl.ds(base, CS)], colpv)

            def vbody(i, cc):
                sl = pl.ds(i * 16, 16)
                rowv[sl] = rowv[sl] + half
                return cc

            lax.fori_loop(0, CS // 16, vbody, 0)
            return pltpu.async_copy(tab_hbm.at[rowv], rows, sem)

        issue(0, rowva, colpva, rowsa, sema)

        def pair(p, car):
            k = p * 2
            db = issue(k + 1, rowvb, colpvb, rowsb, semb)
            pltpu.make_async_copy(tab_hbm.at[rowva], rowsa, sema).wait()
            pltpu.sync_copy(rowsa, acc.at[colpva], add=True)

            @pl.when(p < NPAIR - 1)
            def _():
                issue(k + 2, rowva, colpva, rowsa, sema)

            db.wait()
            pltpu.sync_copy(rowsb, acc.at[colpvb], add=True)
            return car

        lax.fori_loop(0, NPAIR, pair, 0)

    run_pass(y_hbm)
    plsc.subcore_barrier()

    def resc(j, car):
        rbase = sid * TPN + j * ZR
        pltpu.sync_copy(acc.at[pl.ds(rbase, ZR)], rowsa.at[pl.ds(0, ZR)])
        pltpu.sync_copy(dis2_hbm.at[pl.ds(rbase, ZR)], rowsb.at[pl.ds(0, ZR)])

        def vmul(t, cc):
            i = t // 2
            c = (t % 2) * 16
            rowsa[i, pl.ds(c, 16)] = (rowsa[i, pl.ds(c, 16)] *
                                      rowsb[i, pl.ds(c, 16)])
            return cc

        lax.fori_loop(0, ZR * 2, vmul, 0)
        pltpu.sync_copy(rowsa.at[pl.ds(0, ZR)],
                        y2_hbm.at[pl.ds(cid * NP + rbase, ZR)])
        return car

    lax.fori_loop(0, TPN // ZR, resc, 0)
    plsc.subcore_barrier()

    run_pass(y2_hbm)
    plsc.subcore_barrier()

    def wout(j, car):
        pltpu.sync_copy(acc.at[pl.ds(sid * TPN + j * ZR, ZR)],
                        rowsa.at[pl.ds(0, ZR)])
        pltpu.sync_copy(rowsa.at[pl.ds(0, ZR)],
                        out_hbm.at[pl.ds(cid * NP + sid * TPN + j * ZR, ZR)])
        return car

    lax.fori_loop(0, TPN // ZR, wout, 0)


def _conv_call(y, rowp, colx, dis2, zeros2):
    return pl.kernel(
        _conv_body,
        out_type=(
            jax.ShapeDtypeStruct((2 * NP, H), jnp.float32),
            jax.ShapeDtypeStruct((2 * NP, H), jnp.float32),
        ),
        mesh=_mesh,
        scratch_types=[
            pltpu.VMEM((CS,), jnp.int32),
            pltpu.VMEM((CS,), jnp.int32),
            pltpu.VMEM((CS,), jnp.int32),
            pltpu.VMEM((CS,), jnp.int32),
            pltpu.VMEM((CS, H), jnp.float32),
            pltpu.VMEM((CS, H), jnp.float32),
            pltpu.VMEM_SHARED((NP, H), jnp.float32),
            pltpu.SemaphoreType.DMA,
            pltpu.SemaphoreType.DMA,
        ],
        compiler_params=pltpu.CompilerParams(use_tc_tiling_on_sc=False),
    )(y, rowp, colx, dis2, zeros2)


def kernel(features, edge_index, Wu, Wv, Wproj, ID_emb, preference):
    row = edge_index[0]
    col = edge_index[1]
    rowp = jnp.pad(row, (0, EP - E))
    colp_in = jnp.pad(col, (0, EP - E))

    temp = _front(features, Wu.T, Wv.T, Wproj.T, ID_emb)

    zeros1 = jnp.zeros((TPN,), jnp.float32)
    zeros2 = jnp.zeros((ZR, H), jnp.float32)
    colx, degp = _deg(rowp, colp_in, zeros1)

    x0 = jnp.concatenate([preference, temp], axis=0)
    x0 = jnp.pad(x0, ((0, NP - N), (0, 0)))
    x, y, dis, dis2 = _scale(x0, degp.reshape(2, NP))

    u12, _ = _conv_call(y.reshape(2 * NP, H), rowp, colx, dis2, zeros2)

    xhat = _combine(x, u12.reshape(2, NP, H), dis)
    return (xhat[:N], preference)
